# baseline (speedup 1.0000x reference)
import jax
import jax.numpy as jnp
from jax.experimental import pallas as pl
from jax.experimental.pallas import tpu as pltpu

_MARGIN = 0.1
_N = 4096
_BM = 512
_NBLK = _N // _BM


def _loss_body(cos_a, cos_b, mask_a, mask_b, out_ref, acc_ref):
    i = pl.program_id(0)

    @pl.when(i == 0)
    def _init():
        acc_ref[0] = 0.0
        acc_ref[1] = 0.0

    lt_sum = 0.0
    lnt_sum = 0.0
    for c_ref, m_ref in ((cos_a, mask_a), (cos_b, mask_b)):
        c = c_ref[...]
        m = m_ref[...].astype(jnp.float32)
        cm = c * m
        r = jnp.maximum(c - _MARGIN, 0.0)
        rm = r * m
        cnt = jnp.sum(m, axis=1, keepdims=True)
        scm = jnp.sum(cm, axis=1, keepdims=True)
        sr = jnp.sum(r, axis=1, keepdims=True)
        srm = jnp.sum(rm, axis=1, keepdims=True)
        lt_sum += jnp.sum((cnt - scm) / cnt)
        lnt_sum += jnp.sum((sr - srm) / (_N - cnt))
    acc_ref[0] += lt_sum
    acc_ref[1] += lnt_sum

    @pl.when(i == _NBLK - 1)
    def _emit():
        lt_mean = acc_ref[0] / _N
        lnt_mean = acc_ref[1] / _N
        out_ref[0] = (lt_mean + lnt_mean) * 0.5
        out_ref[1] = lt_mean
        out_ref[2] = lnt_mean


def kernel(cos_pred, mask_gt):
    h = _BM // 2
    out = pl.pallas_call(
        _loss_body,
        grid=(_NBLK,),
        in_specs=[
            pl.BlockSpec((h, _N), lambda i: (2 * i, 0)),
            pl.BlockSpec((h, _N), lambda i: (2 * i + 1, 0)),
            pl.BlockSpec((h, _N), lambda i: (2 * i, 0)),
            pl.BlockSpec((h, _N), lambda i: (2 * i + 1, 0)),
        ],
        out_specs=pl.BlockSpec(memory_space=pltpu.SMEM),
        out_shape=jax.ShapeDtypeStruct((3,), jnp.float32),
        scratch_shapes=[pltpu.SMEM((2,), jnp.float32)],
    )(cos_pred, cos_pred, mask_gt, mask_gt)
    return (out[0], out[1], out[2])
